# baseline (speedup 1.0000x reference)
import functools

import jax
import jax.numpy as jnp
from jax import lax
from jax.experimental import pallas as pl
from jax.experimental.pallas import tpu as pltpu
from jax.experimental.pallas import tpu_sc as plsc

_LANES = 16
_PROBE_CORE = 0


def _mm_kernel(x_ref, w_ref, u_ref, v_ref):
    xb = x_ref[...]
    w = w_ref[...]
    c = w.shape[0]
    w1 = w[:, :c]
    w2 = w[:, c:]
    dn = (((1,), (1,)), ((), ()))
    u_ref[...] = lax.dot_general(xb, w1 - w2, dn, preferred_element_type=jnp.float32)
    v_ref[...] = lax.dot_general(xb, w2, dn, preferred_element_type=jnp.float32)


def _node_tables(xt, w, nt):
    g, c = xt.shape
    grid = g // nt
    return pl.pallas_call(
        _mm_kernel,
        grid=(grid,),
        in_specs=[
            pl.BlockSpec((nt, c), lambda i: (i, 0)),
            pl.BlockSpec((c, 2 * c), lambda i: (0, 0)),
        ],
        out_specs=[pl.BlockSpec((nt, c), lambda i: (i, 0))] * 2,
        out_shape=[jax.ShapeDtypeStruct((g, c), jnp.float32)] * 2,
    )(xt, w)


def _make_edge_max(g_pad, c, k, nb):
    info = plsc.get_sparse_core_info()
    nc, ns = info.num_cores, info.num_subcores
    nw = ns if _PROBE_CORE is not None else nc * ns
    npw = g_pad // nw
    nblk = npw // nb
    assert nblk % 2 == 0
    mesh = plsc.VectorSubcoreMesh(core_axis_name="c", subcore_axis_name="s")

    def _entry(u_hbm, v_hbm, ii_hbm, jj_hbm, b_hbm, out_hbm,
               ii_all, jj_all, ur0, vr0, ur1, vr1, b_v, ob0, ob1,
               su0, sv0, su1, sv1, so0, so1):
        if _PROBE_CORE is not None:
            base_blk = lax.axis_index("s") * nblk

            @pl.when(lax.axis_index("c") == _PROBE_CORE)
            def _():
                _worker(u_hbm, v_hbm, ii_hbm, jj_hbm, b_hbm, out_hbm,
                        ii_all, jj_all, ur0, vr0, ur1, vr1, b_v, ob0, ob1,
                        su0, sv0, su1, sv1, so0, so1, base_blk)
        else:
            wid = lax.axis_index("s") * nc + lax.axis_index("c")
            _worker(u_hbm, v_hbm, ii_hbm, jj_hbm, b_hbm, out_hbm,
                    ii_all, jj_all, ur0, vr0, ur1, vr1, b_v, ob0, ob1,
                    su0, sv0, su1, sv1, so0, so1, wid * nblk)

    def _worker(u_hbm, v_hbm, ii_hbm, jj_hbm, b_hbm, out_hbm,
                ii_all, jj_all, ur0, vr0, ur1, vr1, b_v, ob0, ob1,
                su0, sv0, su1, sv1, so0, so1, base_blk):
        pltpu.sync_copy(b_hbm, b_v)
        pltpu.sync_copy(ii_hbm.at[pl.ds(base_blk, nblk)], ii_all)
        pltpu.sync_copy(jj_hbm.at[pl.ds(base_blk, nblk)], jj_all)

        def issue(i, ur, vr, su, sv):
            pltpu.async_copy(u_hbm.at[ii_all.at[i]], ur, su)
            pltpu.async_copy(v_hbm.at[jj_all.at[i]], vr, sv)

        def wait_rows(i, ur, vr, su, sv):
            pltpu.make_async_copy(u_hbm.at[ii_all.at[i]], ur, su).wait()
            pltpu.make_async_copy(v_hbm.at[jj_all.at[i]], vr, sv).wait()

        def out_slice(i):
            return out_hbm.at[pl.ds((base_blk + i) * nb, nb)]

        def compute(ur, vr, ob):
            def node(n, ncarry):
                for c16 in range(c // _LANES):
                    sl = pl.ds(c16 * _LANES, _LANES)
                    acc = ur[n * k, sl] + vr[n * k, sl]
                    for kk in range(1, k):
                        acc = jnp.maximum(acc, ur[n * k + kk, sl] + vr[n * k + kk, sl])
                    ob[n, sl] = jnp.maximum(acc + b_v[sl], 0.0)
                return ncarry
            lax.fori_loop(0, nb, node, 0)

        issue(0, ur0, vr0, su0, sv0)

        def body(i2, carry):
            b0 = 2 * i2
            b1 = b0 + 1
            issue(b1, ur1, vr1, su1, sv1)
            wait_rows(b0, ur0, vr0, su0, sv0)

            @pl.when(i2 > 0)
            def _():
                pltpu.make_async_copy(ob0, out_slice(b0 - 2), so0).wait()
            compute(ur0, vr0, ob0)
            pltpu.async_copy(ob0, out_slice(b0), so0)

            @pl.when(b0 + 2 < nblk)
            def _():
                issue(b0 + 2, ur0, vr0, su0, sv0)
            wait_rows(b1, ur1, vr1, su1, sv1)

            @pl.when(i2 > 0)
            def _():
                pltpu.make_async_copy(ob1, out_slice(b1 - 2), so1).wait()
            compute(ur1, vr1, ob1)
            pltpu.async_copy(ob1, out_slice(b1), so1)
            return carry

        lax.fori_loop(0, nblk // 2, body, 0)
        pltpu.make_async_copy(ob0, out_slice(nblk - 2), so0).wait()
        pltpu.make_async_copy(ob1, out_slice(nblk - 1), so1).wait()

    return functools.partial(
        pl.kernel,
        mesh=mesh,
        out_type=jax.ShapeDtypeStruct((g_pad, c), jnp.float32),
        scratch_types=[
            pltpu.VMEM((nblk, nb * k), jnp.int32),
            pltpu.VMEM((nblk, nb * k), jnp.int32),
            pltpu.VMEM((nb * k, c), jnp.float32),
            pltpu.VMEM((nb * k, c), jnp.float32),
            pltpu.VMEM((nb * k, c), jnp.float32),
            pltpu.VMEM((nb * k, c), jnp.float32),
            pltpu.VMEM((c,), jnp.float32),
            pltpu.VMEM((nb, c), jnp.float32),
            pltpu.VMEM((nb, c), jnp.float32),
            pltpu.SemaphoreType.DMA,
            pltpu.SemaphoreType.DMA,
            pltpu.SemaphoreType.DMA,
            pltpu.SemaphoreType.DMA,
            pltpu.SemaphoreType.DMA,
            pltpu.SemaphoreType.DMA,
        ],
    )(_entry)


def kernel(x, edge_index, W, b):
    bsz, c, n, _ = x.shape
    kk = edge_index.shape[-1]
    g = bsz * n

    xt = jnp.transpose(x[:, :, :, 0], (0, 2, 1)).reshape(g, c)

    u, v = _node_tables(xt, W, nt=2000)

    offs = (jnp.arange(bsz, dtype=jnp.int32) * n)[:, None, None]
    idx_i = (edge_index[1] + offs).reshape(-1)
    idx_j = (edge_index[0] + offs).reshape(-1)

    nb = 8
    nw = 32
    gran = nw * nb * 2
    g_pad = ((g + gran - 1) // gran) * gran
    pad = g_pad - g
    if pad:
        zp = jnp.zeros((pad * kk,), jnp.int32)
        idx_i = jnp.concatenate([idx_i, zp])
        idx_j = jnp.concatenate([idx_j, zp])
    idx_i = idx_i.reshape(g_pad // nb, nb * kk)
    idx_j = idx_j.reshape(g_pad // nb, nb * kk)

    edge_max = _make_edge_max(g_pad, c, kk, nb)
    o_pad = edge_max(u, v, idx_i, idx_j, b)

    out = o_pad[:g].reshape(bsz, n, c).transpose(0, 2, 1)[..., None]
    return out


# speedup vs baseline: 1.0229x; 1.0229x over previous
import functools

import jax
import jax.numpy as jnp
from jax import lax
from jax.experimental import pallas as pl
from jax.experimental.pallas import tpu as pltpu
from jax.experimental.pallas import tpu_sc as plsc

_LANES = 16
_PROBE_CORE = 1


def _mm_kernel(x_ref, w_ref, u_ref, v_ref):
    xb = x_ref[...]
    w = w_ref[...]
    c = w.shape[0]
    w1 = w[:, :c]
    w2 = w[:, c:]
    dn = (((1,), (1,)), ((), ()))
    u_ref[...] = lax.dot_general(xb, w1 - w2, dn, preferred_element_type=jnp.float32)
    v_ref[...] = lax.dot_general(xb, w2, dn, preferred_element_type=jnp.float32)


def _node_tables(xt, w, nt):
    g, c = xt.shape
    grid = g // nt
    return pl.pallas_call(
        _mm_kernel,
        grid=(grid,),
        in_specs=[
            pl.BlockSpec((nt, c), lambda i: (i, 0)),
            pl.BlockSpec((c, 2 * c), lambda i: (0, 0)),
        ],
        out_specs=[pl.BlockSpec((nt, c), lambda i: (i, 0))] * 2,
        out_shape=[jax.ShapeDtypeStruct((g, c), jnp.float32)] * 2,
    )(xt, w)


def _make_edge_max(g_pad, c, k, nb):
    info = plsc.get_sparse_core_info()
    nc, ns = info.num_cores, info.num_subcores
    nw = ns if _PROBE_CORE is not None else nc * ns
    npw = g_pad // nw
    nblk = npw // nb
    assert nblk % 2 == 0
    mesh = plsc.VectorSubcoreMesh(core_axis_name="c", subcore_axis_name="s")

    def _entry(u_hbm, v_hbm, ii_hbm, jj_hbm, b_hbm, out_hbm,
               ii_all, jj_all, ur0, vr0, ur1, vr1, b_v, ob0, ob1,
               su0, sv0, su1, sv1, so0, so1):
        if _PROBE_CORE is not None:
            base_blk = lax.axis_index("s") * nblk

            @pl.when(lax.axis_index("c") == _PROBE_CORE)
            def _():
                _worker(u_hbm, v_hbm, ii_hbm, jj_hbm, b_hbm, out_hbm,
                        ii_all, jj_all, ur0, vr0, ur1, vr1, b_v, ob0, ob1,
                        su0, sv0, su1, sv1, so0, so1, base_blk)
        else:
            wid = lax.axis_index("s") * nc + lax.axis_index("c")
            _worker(u_hbm, v_hbm, ii_hbm, jj_hbm, b_hbm, out_hbm,
                    ii_all, jj_all, ur0, vr0, ur1, vr1, b_v, ob0, ob1,
                    su0, sv0, su1, sv1, so0, so1, wid * nblk)

    def _worker(u_hbm, v_hbm, ii_hbm, jj_hbm, b_hbm, out_hbm,
                ii_all, jj_all, ur0, vr0, ur1, vr1, b_v, ob0, ob1,
                su0, sv0, su1, sv1, so0, so1, base_blk):
        pltpu.sync_copy(b_hbm, b_v)
        pltpu.sync_copy(ii_hbm.at[pl.ds(base_blk, nblk)], ii_all)
        pltpu.sync_copy(jj_hbm.at[pl.ds(base_blk, nblk)], jj_all)

        def issue(i, ur, vr, su, sv):
            pltpu.async_copy(u_hbm.at[ii_all.at[i]], ur, su)
            pltpu.async_copy(v_hbm.at[jj_all.at[i]], vr, sv)

        def wait_rows(i, ur, vr, su, sv):
            pltpu.make_async_copy(u_hbm.at[ii_all.at[i]], ur, su).wait()
            pltpu.make_async_copy(v_hbm.at[jj_all.at[i]], vr, sv).wait()

        def out_slice(i):
            return out_hbm.at[pl.ds((base_blk + i) * nb, nb)]

        def compute(ur, vr, ob):
            def node(n, ncarry):
                for c16 in range(c // _LANES):
                    sl = pl.ds(c16 * _LANES, _LANES)
                    acc = ur[n * k, sl] + vr[n * k, sl]
                    for kk in range(1, k):
                        acc = jnp.maximum(acc, ur[n * k + kk, sl] + vr[n * k + kk, sl])
                    ob[n, sl] = jnp.maximum(acc + b_v[sl], 0.0)
                return ncarry
            lax.fori_loop(0, nb, node, 0)

        issue(0, ur0, vr0, su0, sv0)

        def body(i2, carry):
            b0 = 2 * i2
            b1 = b0 + 1
            issue(b1, ur1, vr1, su1, sv1)
            wait_rows(b0, ur0, vr0, su0, sv0)

            @pl.when(i2 > 0)
            def _():
                pltpu.make_async_copy(ob0, out_slice(b0 - 2), so0).wait()
            compute(ur0, vr0, ob0)
            pltpu.async_copy(ob0, out_slice(b0), so0)

            @pl.when(b0 + 2 < nblk)
            def _():
                issue(b0 + 2, ur0, vr0, su0, sv0)
            wait_rows(b1, ur1, vr1, su1, sv1)

            @pl.when(i2 > 0)
            def _():
                pltpu.make_async_copy(ob1, out_slice(b1 - 2), so1).wait()
            compute(ur1, vr1, ob1)
            pltpu.async_copy(ob1, out_slice(b1), so1)
            return carry

        lax.fori_loop(0, nblk // 2, body, 0)
        pltpu.make_async_copy(ob0, out_slice(nblk - 2), so0).wait()
        pltpu.make_async_copy(ob1, out_slice(nblk - 1), so1).wait()

    return functools.partial(
        pl.kernel,
        mesh=mesh,
        out_type=jax.ShapeDtypeStruct((g_pad, c), jnp.float32),
        scratch_types=[
            pltpu.VMEM((nblk, nb * k), jnp.int32),
            pltpu.VMEM((nblk, nb * k), jnp.int32),
            pltpu.VMEM((nb * k, c), jnp.float32),
            pltpu.VMEM((nb * k, c), jnp.float32),
            pltpu.VMEM((nb * k, c), jnp.float32),
            pltpu.VMEM((nb * k, c), jnp.float32),
            pltpu.VMEM((c,), jnp.float32),
            pltpu.VMEM((nb, c), jnp.float32),
            pltpu.VMEM((nb, c), jnp.float32),
            pltpu.SemaphoreType.DMA,
            pltpu.SemaphoreType.DMA,
            pltpu.SemaphoreType.DMA,
            pltpu.SemaphoreType.DMA,
            pltpu.SemaphoreType.DMA,
            pltpu.SemaphoreType.DMA,
        ],
    )(_entry)


def kernel(x, edge_index, W, b):
    bsz, c, n, _ = x.shape
    kk = edge_index.shape[-1]
    g = bsz * n

    xt = jnp.transpose(x[:, :, :, 0], (0, 2, 1)).reshape(g, c)

    u, v = _node_tables(xt, W, nt=2000)

    offs = (jnp.arange(bsz, dtype=jnp.int32) * n)[:, None, None]
    idx_i = (edge_index[1] + offs).reshape(-1)
    idx_j = (edge_index[0] + offs).reshape(-1)

    nb = 8
    nw = 32
    gran = nw * nb * 2
    g_pad = ((g + gran - 1) // gran) * gran
    pad = g_pad - g
    if pad:
        zp = jnp.zeros((pad * kk,), jnp.int32)
        idx_i = jnp.concatenate([idx_i, zp])
        idx_j = jnp.concatenate([idx_j, zp])
    idx_i = idx_i.reshape(g_pad // nb, nb * kk)
    idx_j = idx_j.reshape(g_pad // nb, nb * kk)

    edge_max = _make_edge_max(g_pad, c, kk, nb)
    o_pad = edge_max(u, v, idx_i, idx_j, b)

    out = o_pad[:g].reshape(bsz, n, c).transpose(0, 2, 1)[..., None]
    return out


# speedup vs baseline: 1.1133x; 1.0884x over previous
import functools

import jax
import jax.numpy as jnp
from jax import lax
from jax.experimental import pallas as pl
from jax.experimental.pallas import tpu as pltpu
from jax.experimental.pallas import tpu_sc as plsc

_LANES = 16
_HI = -65536


def _mm_kernel(x_ref, w_ref, u_ref, v_ref):
    xb = x_ref[...]
    w = w_ref[...]
    c = w.shape[0]
    w1 = w[:, :c]
    w2 = w[:, c:]
    dn = (((1,), (1,)), ((), ()))
    u = lax.dot_general(xb, w1 - w2, dn, preferred_element_type=jnp.float32)
    v = lax.dot_general(xb, w2, dn, preferred_element_type=jnp.float32)
    u_ref[...] = u.astype(jnp.bfloat16)
    v_ref[...] = v.astype(jnp.bfloat16)


def _node_tables(xt, w, nt):
    g, c = xt.shape
    grid = g // nt
    return pl.pallas_call(
        _mm_kernel,
        grid=(grid,),
        in_specs=[
            pl.BlockSpec((nt, c), lambda i: (i, 0)),
            pl.BlockSpec((c, 2 * c), lambda i: (0, 0)),
        ],
        out_specs=[pl.BlockSpec((nt, c), lambda i: (i, 0))] * 2,
        out_shape=[jax.ShapeDtypeStruct((g, c), jnp.bfloat16)] * 2,
    )(xt, w)


def _make_edge_max(g_pad, c, k, nb):
    cw = c // 2
    info = plsc.get_sparse_core_info()
    nc, ns = info.num_cores, info.num_subcores
    nw = nc * ns
    npw = g_pad // nw
    nblk = npw // nb
    assert nblk % 2 == 0
    mesh = plsc.VectorSubcoreMesh(core_axis_name="c", subcore_axis_name="s")

    def _worker(u_hbm, v_hbm, ii_hbm, jj_hbm, b_hbm, out_hbm,
                ii_all, jj_all, ur0, vr0, ur1, vr1, b_v, ob0, ob1,
                su0, sv0, su1, sv1, so0, so1, base_blk):
        pltpu.sync_copy(b_hbm, b_v)
        pltpu.sync_copy(ii_hbm.at[pl.ds(base_blk, nblk)], ii_all)
        pltpu.sync_copy(jj_hbm.at[pl.ds(base_blk, nblk)], jj_all)

        def issue(i, ur, vr, su, sv):
            pltpu.async_copy(u_hbm.at[ii_all.at[i]], ur, su)
            pltpu.async_copy(v_hbm.at[jj_all.at[i]], vr, sv)

        def wait_rows(i, ur, vr, su, sv):
            pltpu.make_async_copy(u_hbm.at[ii_all.at[i]], ur, su).wait()
            pltpu.make_async_copy(v_hbm.at[jj_all.at[i]], vr, sv).wait()

        def out_slice(i):
            return out_hbm.at[pl.ds((base_blk + i) * nb, nb)]

        def widen(w):
            lo = lax.bitcast_convert_type(w << 16, jnp.float32)
            hi = lax.bitcast_convert_type(w & _HI, jnp.float32)
            return lo, hi

        def compute(ur, vr, ob):
            def node(n, ncarry):
                for cc in range(cw // _LANES):
                    sl = pl.ds(cc * _LANES, _LANES)
                    ua, ub = widen(ur[n * k, sl])
                    va, vb = widen(vr[n * k, sl])
                    acc_a = ua + va
                    acc_b = ub + vb
                    for kk in range(1, k):
                        ua, ub = widen(ur[n * k + kk, sl])
                        va, vb = widen(vr[n * k + kk, sl])
                        acc_a = jnp.maximum(acc_a, ua + va)
                        acc_b = jnp.maximum(acc_b, ub + vb)
                    ba = b_v[pl.ds(cc * 2 * _LANES, _LANES)]
                    bb = b_v[pl.ds(cc * 2 * _LANES + _LANES, _LANES)]
                    acc_a = jnp.maximum(acc_a + ba, 0.0)
                    acc_b = jnp.maximum(acc_b + bb, 0.0)
                    ob[n, pl.ds(cc * 2 * _LANES, _LANES)] = acc_a
                    ob[n, pl.ds(cc * 2 * _LANES + _LANES, _LANES)] = acc_b
                return ncarry
            lax.fori_loop(0, nb, node, 0)

        issue(0, ur0, vr0, su0, sv0)

        def body(i2, carry):
            b0 = 2 * i2
            b1 = b0 + 1
            issue(b1, ur1, vr1, su1, sv1)
            wait_rows(b0, ur0, vr0, su0, sv0)

            @pl.when(i2 > 0)
            def _():
                pltpu.make_async_copy(ob0, out_slice(b0 - 2), so0).wait()
            compute(ur0, vr0, ob0)
            pltpu.async_copy(ob0, out_slice(b0), so0)

            @pl.when(b0 + 2 < nblk)
            def _():
                issue(b0 + 2, ur0, vr0, su0, sv0)
            wait_rows(b1, ur1, vr1, su1, sv1)

            @pl.when(i2 > 0)
            def _():
                pltpu.make_async_copy(ob1, out_slice(b1 - 2), so1).wait()
            compute(ur1, vr1, ob1)
            pltpu.async_copy(ob1, out_slice(b1), so1)
            return carry

        lax.fori_loop(0, nblk // 2, body, 0)
        pltpu.make_async_copy(ob0, out_slice(nblk - 2), so0).wait()
        pltpu.make_async_copy(ob1, out_slice(nblk - 1), so1).wait()

    def _entry(u_hbm, v_hbm, ii_hbm, jj_hbm, b_hbm, out_hbm,
               ii_all, jj_all, ur0, vr0, ur1, vr1, b_v, ob0, ob1,
               su0, sv0, su1, sv1, so0, so1):
        wid = lax.axis_index("s") * nc + lax.axis_index("c")
        _worker(u_hbm, v_hbm, ii_hbm, jj_hbm, b_hbm, out_hbm,
                ii_all, jj_all, ur0, vr0, ur1, vr1, b_v, ob0, ob1,
                su0, sv0, su1, sv1, so0, so1, wid * nblk)

    return functools.partial(
        pl.kernel,
        mesh=mesh,
        compiler_params=pltpu.CompilerParams(use_tc_tiling_on_sc=False),
        out_type=jax.ShapeDtypeStruct((g_pad, c), jnp.float32),
        scratch_types=[
            pltpu.VMEM((nblk, nb * k), jnp.int32),
            pltpu.VMEM((nblk, nb * k), jnp.int32),
            pltpu.VMEM((nb * k, cw), jnp.int32),
            pltpu.VMEM((nb * k, cw), jnp.int32),
            pltpu.VMEM((nb * k, cw), jnp.int32),
            pltpu.VMEM((nb * k, cw), jnp.int32),
            pltpu.VMEM((c,), jnp.float32),
            pltpu.VMEM((nb, c), jnp.float32),
            pltpu.VMEM((nb, c), jnp.float32),
            pltpu.SemaphoreType.DMA,
            pltpu.SemaphoreType.DMA,
            pltpu.SemaphoreType.DMA,
            pltpu.SemaphoreType.DMA,
            pltpu.SemaphoreType.DMA,
            pltpu.SemaphoreType.DMA,
        ],
    )(_entry)


def _pack_i32(a_bf16):
    s = a_bf16.shape
    return lax.bitcast_convert_type(
        a_bf16.reshape(s[:-1] + (s[-1] // 2, 2)), jnp.int32)


def kernel(x, edge_index, W, b):
    bsz, c, n, _ = x.shape
    kk = edge_index.shape[-1]
    g = bsz * n

    xt = jnp.transpose(x[:, :, :, 0], (0, 2, 1)).reshape(g, c)

    u, v = _node_tables(xt, W, nt=2000)
    u_p = _pack_i32(u)
    v_p = _pack_i32(v)
    b_perm = b.reshape(c // 32, 16, 2).transpose(0, 2, 1).reshape(c)

    offs = (jnp.arange(bsz, dtype=jnp.int32) * n)[:, None, None]
    idx_i = (edge_index[1] + offs).reshape(-1)
    idx_j = (edge_index[0] + offs).reshape(-1)

    nb = 8
    nw = 32
    gran = nw * nb * 2
    g_pad = ((g + gran - 1) // gran) * gran
    pad = g_pad - g
    if pad:
        zp = jnp.zeros((pad * kk,), jnp.int32)
        idx_i = jnp.concatenate([idx_i, zp])
        idx_j = jnp.concatenate([idx_j, zp])
    idx_i = idx_i.reshape(g_pad // nb, nb * kk)
    idx_j = idx_j.reshape(g_pad // nb, nb * kk)

    edge_max = _make_edge_max(g_pad, c, kk, nb)
    o_pad = edge_max(u_p, v_p, idx_i, idx_j, b_perm)

    o = o_pad[:g].reshape(g, c // 32, 2, 16).transpose(0, 1, 3, 2).reshape(g, c)
    out = o.reshape(bsz, n, c).transpose(0, 2, 1)[..., None]
    return out


# speedup vs baseline: 1.1835x; 1.0631x over previous
import functools

import jax
import jax.numpy as jnp
from jax import lax
from jax.experimental import pallas as pl
from jax.experimental.pallas import tpu as pltpu
from jax.experimental.pallas import tpu_sc as plsc

_LANES = 16


def _mm_kernel(x_ref, w_ref, u_ref, v_ref):
    xb = x_ref[...]
    w = w_ref[...]
    c = w.shape[0]
    w1 = w[:, :c]
    w2 = w[:, c:]
    dn = (((1,), (1,)), ((), ()))
    u_ref[...] = lax.dot_general(xb, w1 - w2, dn, preferred_element_type=jnp.float32)
    v_ref[...] = lax.dot_general(xb, w2, dn, preferred_element_type=jnp.float32)


def _node_tables(xt, w, nt):
    g, c = xt.shape
    grid = g // nt
    return pl.pallas_call(
        _mm_kernel,
        grid=(grid,),
        in_specs=[
            pl.BlockSpec((nt, c), lambda i: (i, 0)),
            pl.BlockSpec((c, 2 * c), lambda i: (0, 0)),
        ],
        out_specs=[pl.BlockSpec((nt, c), lambda i: (i, 0))] * 2,
        out_shape=[jax.ShapeDtypeStruct((g, c), jnp.float32)] * 2,
    )(xt, w)


def _make_edge_max(g_pad, c, k, nb):
    info = plsc.get_sparse_core_info()
    nc, ns = info.num_cores, info.num_subcores
    nw = nc * ns
    npw = g_pad // nw
    nblk = npw // nb
    assert nblk % 2 == 0
    mesh = plsc.VectorSubcoreMesh(core_axis_name="c", subcore_axis_name="s")

    def _worker(u_hbm, v_hbm, ii_hbm, jj_hbm, b_hbm, out_hbm,
                ii_all, jj_all, ur0, vr0, ur1, vr1, b_v, ob0, ob1,
                su0, sv0, su1, sv1, so0, so1, base_blk):
        pltpu.sync_copy(b_hbm, b_v)
        pltpu.sync_copy(ii_hbm.at[pl.ds(base_blk, nblk)], ii_all)
        pltpu.sync_copy(jj_hbm.at[pl.ds(base_blk, nblk)], jj_all)

        def issue(i, ur, vr, su, sv):
            pltpu.async_copy(u_hbm.at[ii_all.at[i]], ur, su)
            pltpu.async_copy(v_hbm.at[jj_all.at[i]], vr, sv)

        def wait_rows(i, ur, vr, su, sv):
            pltpu.make_async_copy(u_hbm.at[ii_all.at[i]], ur, su).wait()
            pltpu.make_async_copy(v_hbm.at[jj_all.at[i]], vr, sv).wait()

        def out_slice(i):
            return out_hbm.at[pl.ds((base_blk + i) * nb, nb)]

        def compute(ur, vr, ob):
            def node(n, ncarry):
                for c16 in range(c // _LANES):
                    sl = pl.ds(c16 * _LANES, _LANES)
                    acc = ur[n * k, sl] + vr[n * k, sl]
                    for kk in range(1, k):
                        acc = jnp.maximum(acc, ur[n * k + kk, sl] + vr[n * k + kk, sl])
                    ob[n, sl] = jnp.maximum(acc + b_v[sl], 0.0)
                return ncarry
            lax.fori_loop(0, nb, node, 0)

        issue(0, ur0, vr0, su0, sv0)

        def body(i2, carry):
            b0 = 2 * i2
            b1 = b0 + 1
            issue(b1, ur1, vr1, su1, sv1)
            wait_rows(b0, ur0, vr0, su0, sv0)

            @pl.when(i2 > 0)
            def _():
                pltpu.make_async_copy(ob0, out_slice(b0 - 2), so0).wait()
            compute(ur0, vr0, ob0)
            pltpu.async_copy(ob0, out_slice(b0), so0)

            @pl.when(b0 + 2 < nblk)
            def _():
                issue(b0 + 2, ur0, vr0, su0, sv0)
            wait_rows(b1, ur1, vr1, su1, sv1)

            @pl.when(i2 > 0)
            def _():
                pltpu.make_async_copy(ob1, out_slice(b1 - 2), so1).wait()
            compute(ur1, vr1, ob1)
            pltpu.async_copy(ob1, out_slice(b1), so1)
            return carry

        lax.fori_loop(0, nblk // 2, body, 0)
        pltpu.make_async_copy(ob0, out_slice(nblk - 2), so0).wait()
        pltpu.make_async_copy(ob1, out_slice(nblk - 1), so1).wait()

    def _entry(u_hbm, v_hbm, ii_hbm, jj_hbm, b_hbm, out_hbm,
               ii_all, jj_all, ur0, vr0, ur1, vr1, b_v, ob0, ob1,
               su0, sv0, su1, sv1, so0, so1):
        wid = lax.axis_index("s") * nc + lax.axis_index("c")
        _worker(u_hbm, v_hbm, ii_hbm, jj_hbm, b_hbm, out_hbm,
                ii_all, jj_all, ur0, vr0, ur1, vr1, b_v, ob0, ob1,
                su0, sv0, su1, sv1, so0, so1, wid * nblk)

    return functools.partial(
        pl.kernel,
        mesh=mesh,
        out_type=jax.ShapeDtypeStruct((g_pad, c), jnp.float32),
        scratch_types=[
            pltpu.VMEM((nblk, nb * k), jnp.int32),
            pltpu.VMEM((nblk, nb * k), jnp.int32),
            pltpu.VMEM((nb * k, c), jnp.float32),
            pltpu.VMEM((nb * k, c), jnp.float32),
            pltpu.VMEM((nb * k, c), jnp.float32),
            pltpu.VMEM((nb * k, c), jnp.float32),
            pltpu.VMEM((c,), jnp.float32),
            pltpu.VMEM((nb, c), jnp.float32),
            pltpu.VMEM((nb, c), jnp.float32),
            pltpu.SemaphoreType.DMA,
            pltpu.SemaphoreType.DMA,
            pltpu.SemaphoreType.DMA,
            pltpu.SemaphoreType.DMA,
            pltpu.SemaphoreType.DMA,
            pltpu.SemaphoreType.DMA,
        ],
    )(_entry)


def kernel(x, edge_index, W, b):
    bsz, c, n, _ = x.shape
    kk = edge_index.shape[-1]
    g = bsz * n

    xt = jnp.transpose(x[:, :, :, 0], (0, 2, 1)).reshape(g, c)

    u, v = _node_tables(xt, W, nt=2000)

    offs = (jnp.arange(bsz, dtype=jnp.int32) * n)[:, None, None]
    idx_i = (edge_index[1] + offs).reshape(-1)
    idx_j = (edge_index[0] + offs).reshape(-1)

    nb = 8
    nw = 32
    gran = nw * nb * 2
    g_pad = ((g + gran - 1) // gran) * gran
    pad = g_pad - g
    if pad:
        zp = jnp.zeros((pad * kk,), jnp.int32)
        idx_i = jnp.concatenate([idx_i, zp])
        idx_j = jnp.concatenate([idx_j, zp])
    idx_i = idx_i.reshape(g_pad // nb, nb * kk)
    idx_j = idx_j.reshape(g_pad // nb, nb * kk)

    edge_max = _make_edge_max(g_pad, c, kk, nb)
    o_pad = edge_max(u, v, idx_i, idx_j, b)

    out = o_pad[:g].reshape(bsz, n, c).transpose(0, 2, 1)[..., None]
    return out
